# baseline (device time: 23633 ns/iter reference)
import jax
import jax.numpy as jnp
from jax import lax
from jax.experimental import pallas as pl
from jax.experimental.pallas import tpu as pltpu

N_DEV = 4
N_TOK = 512
D_IN = 256
D_OUT = 512
N_EXP = 16
EXP_PER_DEV = 4
CHUNK = N_TOK // N_DEV


def kernel(x, router_W, route_idx, expert_W):
    def body(x_ref, rw_ref, idx_ref, ew_ref, out_ref,
             full_ref, acc_ref, comm_ref, send_sems, recv_sems):
        my = lax.axis_index("i")
        left = lax.rem(my + N_DEV - 1, N_DEV)
        right = lax.rem(my + 1, N_DEV)

        barrier_sem = pltpu.get_barrier_semaphore()
        for nbr in (left, right):
            pl.semaphore_signal(barrier_sem, inc=1, device_id=(nbr,),
                                device_id_type=pl.DeviceIdType.MESH)
        pl.semaphore_wait(barrier_sem, 2)

        xv = x_ref[:, :]
        scores = jnp.dot(xv, rw_ref[:, :], preferred_element_type=jnp.float32)
        smax = jnp.max(scores, axis=1, keepdims=True)
        p = jnp.exp(scores - smax)
        probs = p / jnp.sum(p, axis=1, keepdims=True)

        idx0 = idx_ref[:, 0:1]
        idx1 = idx_ref[:, 1:2]
        eids = lax.broadcasted_iota(jnp.int32, (N_TOK, N_EXP), 1)
        top2 = (eids == idx0) | (eids == idx1)
        gated = jnp.where(top2, probs, 0.0)
        gates = gated / jnp.sum(gated, axis=1, keepdims=True)

        partial = jnp.zeros((N_TOK, D_OUT), jnp.float32)
        for le in range(EXP_PER_DEV):
            ge = my * EXP_PER_DEV + le
            w = jnp.sum(jnp.where(eids == ge, gates, 0.0), axis=1,
                        keepdims=True)
            xs = (xv * w).astype(jnp.bfloat16)
            partial = partial + jnp.dot(
                xs, ew_ref[le, :, :].astype(jnp.bfloat16),
                preferred_element_type=jnp.float32)
        full_ref[:, :] = partial

        for j in range(N_DEV):
            c = lax.rem(my + 2 * N_DEV - 1 - j, N_DEV)
            acc_ref[j, :, :] = full_ref[pl.ds(c * CHUNK, CHUNK), :]

        for s in range(N_DEV - 1):
            rdma = pltpu.make_async_remote_copy(
                src_ref=acc_ref.at[s],
                dst_ref=comm_ref.at[s],
                send_sem=send_sems.at[s],
                recv_sem=recv_sems.at[s],
                device_id=(right,),
                device_id_type=pl.DeviceIdType.MESH,
            )
            rdma.start()
            rdma.wait()
            acc_ref[s + 1, :, :] = acc_ref[s + 1, :, :] + comm_ref[s, :, :]

        out_ref[:, :] = acc_ref[N_DEV - 1, :, :]

    return pl.pallas_call(
        body,
        out_shape=jax.ShapeDtypeStruct((CHUNK, D_OUT), jnp.float32),
        in_specs=[pl.BlockSpec(memory_space=pltpu.VMEM)] * 4,
        out_specs=pl.BlockSpec(memory_space=pltpu.VMEM),
        scratch_shapes=[
            pltpu.VMEM((N_TOK, D_OUT), jnp.float32),
            pltpu.VMEM((N_DEV, CHUNK, D_OUT), jnp.float32),
            pltpu.VMEM((N_DEV - 1, CHUNK, D_OUT), jnp.float32),
            pltpu.SemaphoreType.DMA((N_DEV - 1,)),
            pltpu.SemaphoreType.DMA((N_DEV - 1,)),
        ],
        compiler_params=pltpu.CompilerParams(collective_id=0),
    )(x, router_W, route_idx, expert_W)


# device time: 13915 ns/iter; 1.6984x vs baseline; 1.6984x over previous
import jax
import jax.numpy as jnp
from jax import lax
from jax.experimental import pallas as pl
from jax.experimental.pallas import tpu as pltpu

N_DEV = 4
N_TOK = 512
D_IN = 256
D_OUT = 512
N_EXP = 16
EXP_PER_DEV = 4
CHUNK = N_TOK // N_DEV


def kernel(x, router_W, route_idx, expert_W):
    def body(x_ref, rw_ref, idx_ref, ew_ref, out_ref,
             xs_ref, sendbuf_ref, comm_ref, send_sems, recv_sems):
        my = lax.axis_index("i")

        barrier_sem = pltpu.get_barrier_semaphore()
        for o in range(1, N_DEV):
            pl.semaphore_signal(barrier_sem, inc=1,
                                device_id=(lax.rem(my + o, N_DEV),),
                                device_id_type=pl.DeviceIdType.MESH)

        xv = x_ref[:, :]
        scores = jnp.dot(xv, rw_ref[:, :], preferred_element_type=jnp.float32)
        smax = jnp.max(scores, axis=1, keepdims=True)
        p = jnp.exp(scores - smax)
        probs = p / jnp.sum(p, axis=1, keepdims=True)

        idx0 = idx_ref[:, 0:1]
        idx1 = idx_ref[:, 1:2]
        eids = lax.broadcasted_iota(jnp.int32, (N_TOK, N_EXP), 1)
        top2 = (eids == idx0) | (eids == idx1)
        gated = jnp.where(top2, probs, 0.0)
        gates = gated / jnp.sum(gated, axis=1, keepdims=True)

        for le in range(EXP_PER_DEV):
            ge = my * EXP_PER_DEV + le
            w = jnp.sum(jnp.where(eids == ge, gates, 0.0), axis=1,
                        keepdims=True)
            xs_ref[:, le * D_IN:(le + 1) * D_IN] = (xv * w).astype(jnp.bfloat16)

        ewb = ew_ref[:, :, :].astype(jnp.bfloat16)

        def chunk_partial(row0):
            acc = jnp.zeros((CHUNK, D_OUT), jnp.float32)
            for le in range(EXP_PER_DEV):
                xs_c = xs_ref[pl.ds(row0, CHUNK), le * D_IN:(le + 1) * D_IN]
                acc = acc + jnp.dot(xs_c, ewb[le],
                                    preferred_element_type=jnp.float32)
            return acc

        pl.semaphore_wait(barrier_sem, N_DEV - 1)

        send_rdmas = []
        for o in (2, 1, 3):
            dest = lax.rem(my + o, N_DEV)
            slot = 3 - o
            sendbuf_ref[slot, :, :] = chunk_partial(dest * CHUNK).astype(
                jnp.bfloat16)
            rdma = pltpu.make_async_remote_copy(
                src_ref=sendbuf_ref.at[slot],
                dst_ref=comm_ref.at[slot],
                send_sem=send_sems.at[slot],
                recv_sem=recv_sems.at[slot],
                device_id=(dest,),
                device_id_type=pl.DeviceIdType.MESH,
            )
            rdma.start()
            send_rdmas.append(rdma)

        total = chunk_partial(my * CHUNK)

        for j in range(N_DEV - 1):
            recv = pltpu.make_async_remote_copy(
                src_ref=sendbuf_ref.at[j],
                dst_ref=comm_ref.at[j],
                send_sem=send_sems.at[j],
                recv_sem=recv_sems.at[j],
                device_id=(my,),
                device_id_type=pl.DeviceIdType.MESH,
            )
            recv.wait_recv()
            total = total + comm_ref[j, :, :].astype(jnp.float32)

        for rdma in send_rdmas:
            rdma.wait_send()

        out_ref[:, :] = total

    return pl.pallas_call(
        body,
        out_shape=jax.ShapeDtypeStruct((CHUNK, D_OUT), jnp.float32),
        in_specs=[pl.BlockSpec(memory_space=pltpu.VMEM)] * 4,
        out_specs=pl.BlockSpec(memory_space=pltpu.VMEM),
        scratch_shapes=[
            pltpu.VMEM((N_TOK, EXP_PER_DEV * D_IN), jnp.bfloat16),
            pltpu.VMEM((N_DEV - 1, CHUNK, D_OUT), jnp.bfloat16),
            pltpu.VMEM((N_DEV - 1, CHUNK, D_OUT), jnp.bfloat16),
            pltpu.SemaphoreType.DMA((N_DEV - 1,)),
            pltpu.SemaphoreType.DMA((N_DEV - 1,)),
        ],
        compiler_params=pltpu.CompilerParams(collective_id=0),
    )(x, router_W, route_idx, expert_W)
